# baseline (speedup 1.0000x reference)
import functools

import jax
import jax.numpy as jnp
import numpy as np
from jax import lax
from jax.experimental import pallas as pl
from jax.experimental.pallas import tpu as pltpu
from jax.experimental.pallas import tpu_sc as plsc

_NC = 2
_NS = 16


def kernel(inputs, dist_embedding):
    batch, seq = inputs.shape[0], inputs.shape[1]
    emb = dist_embedding.shape[1]
    total = seq * (seq + 1) // 2
    nrows = batch * total
    nw = _NC * _NS
    per_w = nrows // nw
    assert per_w * nw == nrows and per_w % 8 == 0

    chunk = 128
    nfull = per_w // chunk
    tail = per_w - nfull * chunk

    idx_np = np.concatenate(
        [np.arange(n, dtype=np.int32) for n in range(seq, 0, -1)])
    idx_all = jnp.asarray(np.tile(idx_np, batch))

    mesh = plsc.VectorSubcoreMesh(core_axis_name="c", subcore_axis_name="s")

    @functools.partial(
        pl.kernel,
        mesh=mesh,
        out_type=jax.ShapeDtypeStruct((nrows, emb), jnp.float32),
        scratch_types=[
            pltpu.VMEM((per_w,), jnp.int32),
            pltpu.VMEM((chunk, emb), jnp.float32),
            pltpu.SemaphoreType.DMA,
        ],
    )
    def _gather_kernel(table_hbm, idx_hbm, out_hbm, idx_v, rows_v, sem):
        wid = lax.axis_index("s") * _NC + lax.axis_index("c")
        base = wid * per_w
        pltpu.sync_copy(idx_hbm.at[pl.ds(base, per_w)], idx_v)

        def body(i, carry):
            off = i * chunk
            pltpu.async_copy(
                table_hbm.at[idx_v.at[pl.ds(off, chunk)]], rows_v, sem
            ).wait()
            pltpu.sync_copy(rows_v, out_hbm.at[pl.ds(base + off, chunk)])
            return carry

        lax.fori_loop(0, nfull, body, 0)

        toff = nfull * chunk
        pltpu.async_copy(
            table_hbm.at[idx_v.at[pl.ds(toff, tail)]],
            rows_v.at[pl.ds(0, tail)], sem
        ).wait()
        pltpu.sync_copy(
            rows_v.at[pl.ds(0, tail)], out_hbm.at[pl.ds(base + toff, tail)])

    out = _gather_kernel(dist_embedding, idx_all)
    return out.reshape(batch, total, emb)
